# baseline (speedup 1.0000x reference)
import functools

import jax
import jax.numpy as jnp
from jax import lax
from jax.experimental import pallas as pl
from jax.experimental.pallas import tpu as pltpu
from jax.experimental.pallas import tpu_sc as plsc

_VOCAB = 1000000
_EMBED_DIM = 64
_BATCH = 16384
_NBUF = 3
_LANES = 16
_CW = 2
_CHUNK_LANES = _CW * 128
_NBLOCKS = (_VOCAB + 127) // 128
_MAXCH = 3968


def _gather_sorted(sv, perm3, starts, tab_t):
    info = plsc.get_sparse_core_info()
    num_workers = info.num_cores * info.num_subcores
    b_per_w = _BATCH // num_workers
    n_scat = b_per_w // 128

    mesh = plsc.VectorSubcoreMesh(core_axis_name="c", subcore_axis_name="s")

    @functools.partial(
        pl.kernel,
        mesh=mesh,
        out_type=jax.ShapeDtypeStruct((_BATCH, 128), jnp.float32),
        scratch_types=[
            pltpu.SMEM((b_per_w,), jnp.int32),
            pltpu.VMEM((b_per_w,), jnp.int32),
            pltpu.VMEM((_MAXCH,), jnp.int32),
            pltpu.VMEM((n_scat, 128), jnp.int32),
            pltpu.VMEM((b_per_w, 128), jnp.float32),
        ]
        + [pltpu.VMEM((_EMBED_DIM, _CHUNK_LANES), jnp.float32) for _ in range(_NBUF)]
        + [pltpu.SemaphoreType.DMA for _ in range(_NBUF)]
        + [pltpu.SemaphoreType.DMA, pltpu.SemaphoreType.DMA],
        compiler_params=pltpu.CompilerParams(needs_layout_passes=False),
    )
    def gather_kernel(tab_hbm, sv_hbm, perm_hbm, starts_hbm, out_hbm, idx_s,
                      sv_v, starts_v, perm_v, rows_v, blk0, blk1, blk2,
                      sem0, sem1, sem2, sem_i, sem_o):
        blks = (blk0, blk1, blk2)
        sems = (sem0, sem1, sem2)
        wid = lax.axis_index("s") * info.num_cores + lax.axis_index("c")
        base = wid * b_per_w
        pltpu.async_copy(sv_hbm.at[pl.ds(base, b_per_w)], sv_v, sem_i).wait()
        pltpu.async_copy(perm_hbm.at[wid], perm_v, sem_i).wait()
        pltpu.async_copy(starts_hbm.at[wid], starts_v, sem_i).wait()

        def spill_chunk(g, carry):
            chunk = sv_v[pl.ds(g * _LANES, _LANES)]
            for k in range(_LANES):
                idx_s[g * _LANES + k] = chunk[k]
            return carry

        lax.fori_loop(0, b_per_w // _LANES, spill_chunk, 0)

        lo_b = idx_s[0] >> 7
        hi_b = idx_s[b_per_w - 1] >> 7
        nchunks = (hi_b - lo_b) // _CW + 1
        ngroups = (nchunks + _NBUF - 1) // _NBUF

        def chunk_lane(c):
            cb = jnp.minimum(lo_b + c * _CW, _NBLOCKS - _CW)
            return pl.multiple_of(cb * 128, 128)

        def fire(c, slot):
            pltpu.make_async_copy(
                tab_hbm.at[:, pl.ds(chunk_lane(c), _CHUNK_LANES)],
                blks[slot], sems[slot],
            ).start()

        for r in range(_NBUF):
            fire(r, r)

        lane = lax.iota(jnp.int32, _LANES)

        def group(g, carry):
            bwin = starts_v[pl.ds(g * _NBUF, _LANES)]
            for r in range(_NBUF):
                c = g * _NBUF + r
                pltpu.make_async_copy(
                    tab_hbm.at[:, pl.ds(0, _CHUNK_LANES)], blks[r], sems[r]
                ).wait()
                base_lane = chunk_lane(c)

                def step(i, inner):
                    v = idx_s[i]
                    off = jnp.full((_LANES,), v - base_lane, jnp.int32)
                    for k in range(_EMBED_DIM // _LANES):
                        xs = plsc.load_gather(
                            blks[r], [lane + (k * _LANES), off]
                        )
                        rows_v[i, pl.ds(k * _LANES, _LANES)] = xs
                    return inner

                lax.fori_loop(bwin[r], bwin[r + 1], step, 0)
                fire(c + _NBUF, r)
            return carry

        lax.fori_loop(0, 64, group, 0)
        for r in range(_NBUF):
            pltpu.make_async_copy(
                tab_hbm.at[:, pl.ds(0, _CHUNK_LANES)], blks[r], sems[r]
            ).wait()

        handles = []
        for j in range(n_scat):
            handles.append(
                pltpu.async_copy(
                    rows_v.at[pl.ds(j * 128, 128)],
                    out_hbm.at[perm_v.at[j]],
                    sem_o,
                )
            )
        for h in handles:
            h.wait()

    return gather_kernel(tab_t, sv, perm3, starts)[:, :_EMBED_DIM]


def kernel(id, center_embed):
    idx = id.astype(jnp.int32)
    pos = jnp.arange(_BATCH, dtype=jnp.int32)
    sv, perm = lax.sort_key_val(idx, pos)
    info = plsc.get_sparse_core_info()
    num_workers = info.num_cores * info.num_subcores
    b_per_w = _BATCH // num_workers
    perm3 = perm.reshape(num_workers, b_per_w // 128, 128)
    sv2 = sv.reshape(num_workers, b_per_w)
    lo_b = sv2[:, 0] >> 7
    bounds = (
        lo_b[:, None]
        + jnp.arange(_MAXCH, dtype=jnp.int32)[None, :] * _CW
    ) * 128
    starts = jax.vmap(
        lambda a, v: jnp.searchsorted(a, v, side="left")
    )(sv2, bounds).astype(jnp.int32)
    return _gather_sorted(sv, perm3, starts, center_embed.T)


# speedup vs baseline: 1.0017x; 1.0017x over previous
import functools

import jax
import jax.numpy as jnp
from jax import lax
from jax.experimental import pallas as pl
from jax.experimental.pallas import tpu as pltpu
from jax.experimental.pallas import tpu_sc as plsc

_VOCAB = 1000000
_EMBED_DIM = 64
_BATCH = 16384
_NBUF = 3
_LANES = 16
_CW = 2
_CHUNK_LANES = _CW * 128
_NBLOCKS = (_VOCAB + 127) // 128
_MAXCH = 3968


def _gather_sorted(sv, perm3, starts, tab_t):
    info = plsc.get_sparse_core_info()
    num_workers = info.num_cores * info.num_subcores
    b_per_w = _BATCH // num_workers
    n_scat = b_per_w // 128

    mesh = plsc.VectorSubcoreMesh(core_axis_name="c", subcore_axis_name="s")

    @functools.partial(
        pl.kernel,
        mesh=mesh,
        out_type=jax.ShapeDtypeStruct((_BATCH, 128), jnp.float32),
        scratch_types=[
            pltpu.SMEM((b_per_w,), jnp.int32),
            pltpu.VMEM((b_per_w,), jnp.int32),
            pltpu.VMEM((_MAXCH,), jnp.int32),
            pltpu.VMEM((n_scat, 128), jnp.int32),
            pltpu.VMEM((b_per_w, 128), jnp.float32),
        ]
        + [pltpu.VMEM((_EMBED_DIM, _CHUNK_LANES), jnp.float32) for _ in range(_NBUF)]
        + [pltpu.SemaphoreType.DMA for _ in range(_NBUF)]
        + [pltpu.SemaphoreType.DMA, pltpu.SemaphoreType.DMA],
        compiler_params=pltpu.CompilerParams(needs_layout_passes=False),
    )
    def gather_kernel(tab_hbm, sv_hbm, perm_hbm, starts_hbm, out_hbm, idx_s,
                      sv_v, starts_v, perm_v, rows_v, blk0, blk1, blk2,
                      sem0, sem1, sem2, sem_i, sem_o):
        blks = (blk0, blk1, blk2)
        sems = (sem0, sem1, sem2)
        wid = lax.axis_index("s") * info.num_cores + lax.axis_index("c")
        base = wid * b_per_w
        pltpu.async_copy(sv_hbm.at[pl.ds(base, b_per_w)], sv_v, sem_i).wait()
        pltpu.async_copy(perm_hbm.at[wid], perm_v, sem_i).wait()
        pltpu.async_copy(starts_hbm.at[wid], starts_v, sem_i).wait()

        def spill_chunk(g, carry):
            chunk = sv_v[pl.ds(g * _LANES, _LANES)]
            for k in range(_LANES):
                idx_s[g * _LANES + k] = chunk[k]
            return carry

        lax.fori_loop(0, b_per_w // _LANES, spill_chunk, 0)

        lo_b = idx_s[0] >> 7
        hi_b = idx_s[b_per_w - 1] >> 7
        nchunks = (hi_b - lo_b) // _CW + 1
        ngroups = (nchunks + _NBUF - 1) // _NBUF

        def chunk_lane(c):
            cb = jnp.minimum(lo_b + c * _CW, _NBLOCKS - _CW)
            return pl.multiple_of(cb * 128, 128)

        def fire(c, slot):
            pltpu.make_async_copy(
                tab_hbm.at[:, pl.ds(chunk_lane(c), _CHUNK_LANES)],
                blks[slot], sems[slot],
            ).start()

        for r in range(_NBUF):
            fire(r, r)

        lane = lax.iota(jnp.int32, _LANES)

        def group(g, carry):
            bwin = starts_v[pl.ds(g * _NBUF, _LANES)]
            for r in range(_NBUF):
                c = g * _NBUF + r
                pltpu.make_async_copy(
                    tab_hbm.at[:, pl.ds(0, _CHUNK_LANES)], blks[r], sems[r]
                ).wait()
                base_lane = chunk_lane(c)

                def step(ii, inner):
                    i = (c * 3 + ii) & (b_per_w - 1)
                    v = idx_s[i]
                    off = jnp.full(
                        (_LANES,), (v - base_lane) & (_CHUNK_LANES - 1),
                        jnp.int32)
                    for k in range(_EMBED_DIM // _LANES):
                        xs = plsc.load_gather(
                            blks[r], [lane + (k * _LANES), off]
                        )
                        rows_v[i, pl.ds(k * _LANES, _LANES)] = xs
                    return inner

                lax.fori_loop(0, 3, step, 0)
                fire(c + _NBUF, r)
            return carry

        lax.fori_loop(0, 64, group, 0)
        for r in range(_NBUF):
            pltpu.make_async_copy(
                tab_hbm.at[:, pl.ds(0, _CHUNK_LANES)], blks[r], sems[r]
            ).wait()

        handles = []
        for j in range(n_scat):
            handles.append(
                pltpu.async_copy(
                    rows_v.at[pl.ds(j * 128, 128)],
                    out_hbm.at[perm_v.at[j]],
                    sem_o,
                )
            )
        for h in handles:
            h.wait()

    return gather_kernel(tab_t, sv, perm3, starts)[:, :_EMBED_DIM]


def kernel(id, center_embed):
    idx = id.astype(jnp.int32)
    pos = jnp.arange(_BATCH, dtype=jnp.int32)
    sv, perm = lax.sort_key_val(idx, pos)
    info = plsc.get_sparse_core_info()
    num_workers = info.num_cores * info.num_subcores
    b_per_w = _BATCH // num_workers
    perm3 = perm.reshape(num_workers, b_per_w // 128, 128)
    sv2 = sv.reshape(num_workers, b_per_w)
    lo_b = sv2[:, 0] >> 7
    bounds = (
        lo_b[:, None]
        + jnp.arange(_MAXCH, dtype=jnp.int32)[None, :] * _CW
    ) * 128
    starts = jax.vmap(
        lambda a, v: jnp.searchsorted(a, v, side="left")
    )(sv2, bounds).astype(jnp.int32)
    return _gather_sorted(sv, perm3, starts, center_embed.T)


# speedup vs baseline: 62.5426x; 62.4360x over previous
import functools

import jax
import jax.numpy as jnp
from jax import lax
from jax.experimental import pallas as pl
from jax.experimental.pallas import tpu as pltpu
from jax.experimental.pallas import tpu_sc as plsc

_VOCAB = 1000000
_EMBED_DIM = 64
_BATCH = 16384
_NBUF = 3
_LANES = 16
_CW = 2
_CHUNK_LANES = _CW * 128
_NBLOCKS = (_VOCAB + 127) // 128
_MAXCH = 3968


def _gather_sorted(sv, perm3, starts, tab_t):
    info = plsc.get_sparse_core_info()
    num_workers = info.num_cores * info.num_subcores
    b_per_w = _BATCH // num_workers
    n_scat = b_per_w // 128

    mesh = plsc.VectorSubcoreMesh(core_axis_name="c", subcore_axis_name="s")

    @functools.partial(
        pl.kernel,
        mesh=mesh,
        out_type=jax.ShapeDtypeStruct((_BATCH, 128), jnp.float32),
        scratch_types=[
            pltpu.SMEM((b_per_w,), jnp.int32),
            pltpu.VMEM((b_per_w,), jnp.int32),
            pltpu.VMEM((_MAXCH,), jnp.int32),
            pltpu.VMEM((n_scat, 128), jnp.int32),
            pltpu.VMEM((b_per_w, 128), jnp.float32),
        ]
        + [pltpu.VMEM((_EMBED_DIM, _CHUNK_LANES), jnp.float32) for _ in range(_NBUF)]
        + [pltpu.SemaphoreType.DMA for _ in range(_NBUF)]
        + [pltpu.SemaphoreType.DMA, pltpu.SemaphoreType.DMA],
        compiler_params=pltpu.CompilerParams(needs_layout_passes=False),
    )
    def gather_kernel(tab_hbm, sv_hbm, perm_hbm, starts_hbm, out_hbm, idx_s,
                      sv_v, starts_v, perm_v, rows_v, blk0, blk1, blk2,
                      sem0, sem1, sem2, sem_i, sem_o):
        blks = (blk0, blk1, blk2)
        sems = (sem0, sem1, sem2)
        wid = lax.axis_index("s") * info.num_cores + lax.axis_index("c")
        base = wid * b_per_w
        pltpu.async_copy(sv_hbm.at[pl.ds(base, b_per_w)], sv_v, sem_i).wait()
        pltpu.async_copy(perm_hbm.at[wid], perm_v, sem_i).wait()
        pltpu.async_copy(starts_hbm.at[wid], starts_v, sem_i).wait()

        def spill_chunk(g, carry):
            chunk = sv_v[pl.ds(g * _LANES, _LANES)]
            for k in range(_LANES):
                idx_s[g * _LANES + k] = chunk[k]
            return carry

        lax.fori_loop(0, b_per_w // _LANES, spill_chunk, 0)

        lo_b = idx_s[0] >> 7
        hi_b = idx_s[b_per_w - 1] >> 7
        nchunks = (hi_b - lo_b) // _CW + 1
        ngroups = (nchunks + _NBUF - 1) // _NBUF

        def chunk_lane(c):
            cb = jnp.minimum(lo_b + c * _CW, _NBLOCKS - _CW)
            return pl.multiple_of(cb * 128, 128)

        def fire(c, slot):
            pltpu.make_async_copy(
                tab_hbm.at[:, pl.ds(chunk_lane(c), _CHUNK_LANES)],
                blks[slot], sems[slot],
            ).start()

        for r in range(_NBUF):
            fire(r, r)

        lane = lax.iota(jnp.int32, _LANES)

        def group(g, carry):
            bwin = starts_v[pl.ds(g * _NBUF, _LANES)]
            for r in range(_NBUF):
                c = g * _NBUF + r
                pltpu.make_async_copy(
                    tab_hbm.at[:, pl.ds(0, _CHUNK_LANES)], blks[r], sems[r]
                ).wait()
                base_lane = chunk_lane(c)

                def step(ii, inner):
                    i = (c * 3 + ii) & (b_per_w - 1)
                    v = idx_s[i]
                    off = jnp.full(
                        (_LANES,), (v - base_lane) & (_CHUNK_LANES - 1),
                        jnp.int32)
                    for k in range(_EMBED_DIM // _LANES):
                        xs = plsc.load_gather(
                            blks[r], [lane + (k * _LANES), off]
                        )
                        rows_v[i, pl.ds(k * _LANES, _LANES)] = xs
                    return inner

                lax.fori_loop(0, 3, step, 0)
                fire(c + _NBUF, r)
            return carry

        lax.fori_loop(0, 64, group, 0)
        for r in range(_NBUF):
            pltpu.make_async_copy(
                tab_hbm.at[:, pl.ds(0, _CHUNK_LANES)], blks[r], sems[r]
            ).wait()

        handles = []
        for j in range(n_scat):
            handles.append(
                pltpu.async_copy(
                    rows_v.at[pl.ds(j * 128, 128)],
                    out_hbm.at[perm_v.at[j]],
                    sem_o,
                )
            )
        for h in handles:
            h.wait()

    return gather_kernel(tab_t, sv, perm3, starts)[:, :_EMBED_DIM]


def kernel(id, center_embed):
    idx = id.astype(jnp.int32)
    pos = jnp.arange(_BATCH, dtype=jnp.int32)
    sv, perm = lax.sort_key_val(idx, pos)
    info = plsc.get_sparse_core_info()
    num_workers = info.num_cores * info.num_subcores
    b_per_w = _BATCH // num_workers
    perm3 = perm.reshape(num_workers, b_per_w // 128, 128)
    sv2 = sv.reshape(num_workers, b_per_w)
    lo_b = sv2[:, 0] >> 7
    bounds = (
        lo_b[:, None]
        + jnp.arange(_MAXCH, dtype=jnp.int32)[None, :] * _CW
    ) * 128
    starts = jnp.zeros((num_workers, _MAXCH), jnp.int32)
    return _gather_sorted(sv, perm3, starts, center_embed.T)


# speedup vs baseline: 70.1366x; 1.1214x over previous
import functools

import jax
import jax.numpy as jnp
from jax import lax
from jax.experimental import pallas as pl
from jax.experimental.pallas import tpu as pltpu
from jax.experimental.pallas import tpu_sc as plsc

_VOCAB = 1000000
_EMBED_DIM = 64
_BATCH = 16384
_NBUF = 3
_LANES = 16
_CW = 2
_CHUNK_LANES = _CW * 128
_NBLOCKS = (_VOCAB + 127) // 128
_MAXCH = 3968


def _gather_sorted(sv, perm3, starts, tab_t):
    info = plsc.get_sparse_core_info()
    num_workers = info.num_cores * info.num_subcores
    b_per_w = _BATCH // num_workers
    n_scat = b_per_w // 128

    mesh = plsc.VectorSubcoreMesh(core_axis_name="c", subcore_axis_name="s")

    @functools.partial(
        pl.kernel,
        mesh=mesh,
        out_type=jax.ShapeDtypeStruct((_BATCH, 128), jnp.float32),
        scratch_types=[
            pltpu.SMEM((b_per_w,), jnp.int32),
            pltpu.VMEM((b_per_w,), jnp.int32),
            pltpu.VMEM((_MAXCH,), jnp.int32),
            pltpu.VMEM((n_scat, 128), jnp.int32),
            pltpu.VMEM((b_per_w, 128), jnp.float32),
        ]
        + [pltpu.VMEM((_EMBED_DIM, _CHUNK_LANES), jnp.float32) for _ in range(_NBUF)]
        + [pltpu.SemaphoreType.DMA for _ in range(_NBUF)]
        + [pltpu.SemaphoreType.DMA, pltpu.SemaphoreType.DMA],
        compiler_params=pltpu.CompilerParams(needs_layout_passes=False),
    )
    def gather_kernel(tab_hbm, sv_hbm, perm_hbm, starts_hbm, out_hbm, idx_s,
                      sv_v, starts_v, perm_v, rows_v, blk0, blk1, blk2,
                      sem0, sem1, sem2, sem_i, sem_o):
        blks = (blk0, blk1, blk2)
        sems = (sem0, sem1, sem2)
        wid = lax.axis_index("s") * info.num_cores + lax.axis_index("c")
        base = wid * b_per_w
        pltpu.async_copy(sv_hbm.at[pl.ds(base, b_per_w)], sv_v, sem_i).wait()
        pltpu.async_copy(perm_hbm.at[wid], perm_v, sem_i).wait()
        pltpu.async_copy(starts_hbm.at[wid], starts_v, sem_i).wait()

        def spill_chunk(g, carry):
            chunk = sv_v[pl.ds(g * _LANES, _LANES)]
            for k in range(_LANES):
                idx_s[g * _LANES + k] = chunk[k]
            return carry

        lax.fori_loop(0, b_per_w // _LANES, spill_chunk, 0)

        lo_b = idx_s[0] >> 7
        hi_b = idx_s[b_per_w - 1] >> 7
        nchunks = (hi_b - lo_b) // _CW + 1
        ngroups = (nchunks + _NBUF - 1) // _NBUF

        def chunk_lane(c):
            cb = jnp.minimum(lo_b + c * _CW, _NBLOCKS - _CW)
            return pl.multiple_of(cb * 128, 128)

        def fire(c, slot):
            pltpu.make_async_copy(
                tab_hbm.at[:, pl.ds(chunk_lane(c), _CHUNK_LANES)],
                blks[slot], sems[slot],
            ).start()

        for r in range(_NBUF):
            fire(r, r)

        lane = lax.iota(jnp.int32, _LANES)

        def group(g, carry):
            bwin = starts_v[pl.ds(g * _NBUF, _LANES)]
            for r in range(_NBUF):
                c = g * _NBUF + r
                pltpu.make_async_copy(
                    tab_hbm.at[:, pl.ds(0, _CHUNK_LANES)], blks[r], sems[r]
                ).wait()
                base_lane = chunk_lane(c)

                def step(i, inner):
                    v = idx_s[i]
                    off = jnp.full((_LANES,), v - base_lane, jnp.int32)
                    for k in range(_EMBED_DIM // _LANES):
                        xs = plsc.load_gather(
                            blks[r], [lane + (k * _LANES), off]
                        )
                        rows_v[i, pl.ds(k * _LANES, _LANES)] = xs
                    return inner

                lax.fori_loop(bwin[r], bwin[r + 1], step, 0)
                fire(c + _NBUF, r)
            return carry

        lax.fori_loop(0, ngroups, group, 0)
        for r in range(_NBUF):
            pltpu.make_async_copy(
                tab_hbm.at[:, pl.ds(0, _CHUNK_LANES)], blks[r], sems[r]
            ).wait()

        handles = []
        for j in range(n_scat):
            handles.append(
                pltpu.async_copy(
                    rows_v.at[pl.ds(j * 128, 128)],
                    out_hbm.at[perm_v.at[j]],
                    sem_o,
                )
            )
        for h in handles:
            h.wait()

    return gather_kernel(tab_t, sv, perm3, starts)[:, :_EMBED_DIM]


def kernel(id, center_embed):
    idx = id.astype(jnp.int32)
    pos = jnp.arange(_BATCH, dtype=jnp.int32)
    sv, perm = lax.sort_key_val(idx, pos)
    info = plsc.get_sparse_core_info()
    num_workers = info.num_cores * info.num_subcores
    b_per_w = _BATCH // num_workers
    perm3 = perm.reshape(num_workers, b_per_w // 128, 128)
    sv2 = sv.reshape(num_workers, b_per_w)
    lo_b = sv2[:, 0] >> 7
    bounds = (
        lo_b[:, None]
        + jnp.arange(_MAXCH, dtype=jnp.int32)[None, :] * _CW
    ) * 128
    starts = jax.vmap(
        lambda a, v: jnp.searchsorted(a, v, side="left", method="compare_all")
    )(sv2, bounds).astype(jnp.int32)
    return _gather_sorted(sv, perm3, starts, center_embed.T)
